# baseline (speedup 1.0000x reference)
import functools
import math

import jax
import jax.numpy as jnp
from jax import lax
from jax.experimental import pallas as pl
from jax.experimental.pallas import tpu as pltpu
from jax.experimental.pallas import tpu_sc as plsc

N_NODES = 10000
N_PAD = 10240
N_EDGES = 320000
D = 128

NC = 2
NS = 16
NW = NC * NS
E_PER_W = N_EDGES // NW
CHUNK = 128
N_CHUNKS = E_PER_W // CHUNK
TAIL = E_PER_W - N_CHUNKS * CHUNK
ROWS_PER_TILE = N_PAD // NS

_mesh = plsc.VectorSubcoreMesh(core_axis_name="c", subcore_axis_name="s")


CH = 80
NCH = E_PER_W // CH
DEG_K = 25
N_STEADY = 120

_AGG_SCRATCH = [
    pltpu.VMEM_SHARED((N_PAD, D), jnp.float32),
    pltpu.VMEM((NCH, CH), jnp.int32),
    [pltpu.VMEM((CH,), jnp.int32) for _ in range(4)],
    [pltpu.VMEM((CH, D), jnp.float32) for _ in range(3)],
    [pltpu.SemaphoreType.DMA for _ in range(4)],
    [pltpu.SemaphoreType.DMA for _ in range(2)],
    [pltpu.SemaphoreType.DMA for _ in range(2)],
]


def _load_didx2(edge_hbm, didx2, base, sem):
    dbase = N_EDGES + base
    for lo in range(0, NCH, DEG_K):
        hi = min(lo + DEG_K, NCH)
        cps = [pltpu.async_copy(edge_hbm.at[pl.ds(dbase + r * CH, CH)],
                                didx2.at[r], sem)
               for r in range(lo, hi)]
        for cp in cps:
            cp.wait()


def _zero_acc(zeros_hbm, acc_sh, r0):
    pltpu.sync_copy(zeros_hbm, acc_sh.at[pl.ds(r0, ROWS_PER_TILE)])


def _agg_edge_loop(edge_hbm, x_hbm, acc_sh, didx2, sib, rows,
                   isem, gsem, ssem, base):
    def idxload(i, u4):
        pltpu.async_copy(edge_hbm.at[pl.ds(base + i * CH, CH)],
                         sib[u4], isem[u4])

    def idxwait(i, u4):
        pltpu.make_async_copy(edge_hbm.at[pl.ds(base + i * CH, CH)],
                              sib[u4], isem[u4]).wait()

    def gissue(u4, u3, u2):
        pltpu.async_copy(x_hbm.at[sib[u4]], rows[u3], gsem[u2])

    def gwait(u4, u3, u2):
        pltpu.make_async_copy(x_hbm.at[sib[u4]], rows[u3], gsem[u2]).wait()

    def sissue(i, u3, u2):
        pltpu.async_copy(rows[u3], acc_sh.at[didx2.at[i]], ssem[u2], add=True)

    def swait(i, u3, u2):
        pltpu.make_async_copy(rows[u3], acc_sh.at[didx2.at[i]],
                              ssem[u2]).wait()

    idxload(0, 0)
    idxload(1, 1)

    def body(t, carry):
        for u in range(12):
            j = t * 12 + u
            u4, u3, u2 = u % 4, u % 3, u % 2

            @pl.when(j >= 3)
            def _():
                swait(j - 3, u % 3, (u + 1) % 2)

            idxwait(j, u4)
            gissue(u4, u3, u2)

            @pl.when(j + 2 < NCH)
            def _():
                idxload(j + 2, (u + 2) % 4)

            @pl.when(j >= 1)
            def _():
                gwait((u + 3) % 4, (u + 2) % 3, (u + 1) % 2)
                sissue(j - 1, (u + 2) % 3, (u + 1) % 2)
        return carry

    lax.fori_loop(0, N_STEADY // 12, body, 0)

    for j in range(N_STEADY, NCH):
        u = j % 12
        u4, u3, u2 = u % 4, u % 3, u % 2
        swait(j - 3, u % 3, (u + 1) % 2)
        idxwait(j, u4)
        gissue(u4, u3, u2)
        if j + 2 < NCH:
            idxload(j + 2, (u + 2) % 4)
        gwait((u + 3) % 4, (u + 2) % 3, (u + 1) % 2)
        sissue(j - 1, (u + 2) % 3, (u + 1) % 2)
    uL = (NCH - 1) % 12
    gwait(uL % 4, uL % 3, uL % 2)
    sissue(NCH - 1, uL % 3, uL % 2)
    for j in range(NCH - 3, NCH):
        u = j % 12
        swait(j, u % 3, u % 2)


@functools.partial(
    pl.kernel,
    out_type=jax.ShapeDtypeStruct((NC * N_PAD, D), jnp.float32),
    mesh=_mesh,
    scratch_types=_AGG_SCRATCH,
)
def _agg_kernel(edge_hbm, x_hbm, zeros_hbm, acc_out,
                acc_sh, didx2, sib, rows, isem, gsem, ssem):
    c = lax.axis_index("c")
    s = lax.axis_index("s")
    wid = s * NC + c
    r0 = s * ROWS_PER_TILE
    base = wid * E_PER_W

    _zero_acc(zeros_hbm, acc_sh, r0)
    _load_didx2(edge_hbm, didx2, base, isem[3])
    plsc.subcore_barrier()
    _agg_edge_loop(edge_hbm, x_hbm, acc_sh, didx2, sib, rows,
                   isem, gsem, ssem, base)
    plsc.subcore_barrier()
    pltpu.sync_copy(acc_sh.at[pl.ds(r0, ROWS_PER_TILE)],
                    acc_out.at[pl.ds(c * N_PAD + r0, ROWS_PER_TILE)])


@functools.partial(
    pl.kernel,
    out_type=[jax.ShapeDtypeStruct((NC * N_PAD, D), jnp.float32),
              jax.ShapeDtypeStruct((NC * N_PAD, D), jnp.float32)],
    mesh=_mesh,
    scratch_types=_AGG_SCRATCH,
)
def _deg_agg_kernel(edge_hbm, x_hbm, ones_hbm, zeros_hbm,
                    deg_out, acc_out,
                    acc_sh, didx2, sib, rows, isem, gsem, ssem):
    c = lax.axis_index("c")
    s = lax.axis_index("s")
    wid = s * NC + c
    r0 = s * ROWS_PER_TILE
    base = wid * E_PER_W

    _zero_acc(zeros_hbm, acc_sh, r0)
    _load_didx2(edge_hbm, didx2, base, isem[3])
    pltpu.sync_copy(ones_hbm, rows[0])
    plsc.subcore_barrier()
    prev = None
    for bi, lo in enumerate(range(0, NCH, DEG_K)):
        hi = min(lo + DEG_K, NCH)
        cps = [pltpu.async_copy(rows[0], acc_sh.at[didx2.at[r]],
                                ssem[bi % 2], add=True)
               for r in range(lo, hi)]
        if prev is not None:
            for cp in prev:
                cp.wait()
        prev = cps
    for cp in prev:
        cp.wait()
    plsc.subcore_barrier()
    pltpu.sync_copy(acc_sh.at[pl.ds(r0, ROWS_PER_TILE)],
                    deg_out.at[pl.ds(c * N_PAD + r0, ROWS_PER_TILE)])
    plsc.subcore_barrier()

    _zero_acc(zeros_hbm, acc_sh, r0)
    plsc.subcore_barrier()
    _agg_edge_loop(edge_hbm, x_hbm, acc_sh, didx2, sib, rows,
                   isem, gsem, ssem, base)
    plsc.subcore_barrier()
    pltpu.sync_copy(acc_sh.at[pl.ds(r0, ROWS_PER_TILE)],
                    acc_out.at[pl.ds(c * N_PAD + r0, ROWS_PER_TILE)])


N_SUM_ROWS = 2048
L_CTX = 50
IDX_PER_TILE = N_SUM_ROWS * L_CTX // NW
GROUP = 4
N_PAIRS = 8


@functools.partial(
    pl.kernel,
    out_type=jax.ShapeDtypeStruct((N_SUM_ROWS, D), jnp.float32),
    mesh=_mesh,
    scratch_types=[
        pltpu.VMEM((IDX_PER_TILE,), jnp.int32),
        [pltpu.VMEM((GROUP * L_CTX, D), jnp.float32) for _ in range(2)],
        pltpu.VMEM((2 * GROUP, D), jnp.float32),
        [pltpu.SemaphoreType.DMA for _ in range(2)],
    ],
)
def _gsum_kernel(idx_hbm, x_hbm, out_hbm, idxv, rows, outv, sem):
    c = lax.axis_index("c")
    s = lax.axis_index("s")
    wid = s * NC + c
    pltpu.sync_copy(idx_hbm.at[pl.ds(wid * IDX_PER_TILE, IDX_PER_TILE)], idxv)

    def issue(h, b):
        ib = h * (GROUP * L_CTX)
        pltpu.async_copy(x_hbm.at[idxv.at[pl.ds(ib, 128)]],
                         rows[b].at[pl.ds(0, 128)], sem[b])
        pltpu.async_copy(x_hbm.at[idxv.at[pl.ds(ib + 128, 72)]],
                         rows[b].at[pl.ds(128, 72)], sem[b])

    def drain(h, b):
        ib = h * (GROUP * L_CTX)
        pltpu.make_async_copy(x_hbm.at[idxv.at[pl.ds(ib, 128)]],
                              rows[b].at[pl.ds(0, 128)], sem[b]).wait()
        pltpu.make_async_copy(x_hbm.at[idxv.at[pl.ds(ib + 128, 72)]],
                              rows[b].at[pl.ds(128, 72)], sem[b]).wait()

    def reduce(b, half):
        for o in range(GROUP):
            def jbody(j, accs):
                new = accs
                for u in range(5):
                    r = o * L_CTX + j * 5 + u
                    new = tuple(new[k] + rows[b][r, pl.ds(k * 16, 16)]
                                for k in range(8))
                return new
            init = tuple(jnp.zeros((16,), jnp.float32) for _ in range(8))
            accs = lax.fori_loop(0, 10, jbody, init)
            for k in range(8):
                outv[half * GROUP + o, pl.ds(k * 16, 16)] = accs[k]

    issue(0, 0)

    def body(p, carry):
        issue(2 * p + 1, 1)
        drain(2 * p, 0)
        reduce(0, 0)

        @pl.when(p < N_PAIRS - 1)
        def _():
            issue(2 * p + 2, 0)

        drain(2 * p + 1, 1)
        reduce(1, 1)
        pltpu.sync_copy(outv,
                        out_hbm.at[pl.ds(wid * 64 + p * (2 * GROUP),
                                         2 * GROUP)])
        return carry

    lax.fori_loop(0, N_PAIRS, body, 0)


_BLK = 1280
_GRID = N_PAD // _BLK


def _sage1_body(acc_ref, deg_ref, emb_ref, wl_ref, wr_ref, b_ref, o_ref):
    agg = acc_ref[0] + acc_ref[1]
    deg = deg_ref[0] + deg_ref[1]
    inv = 1.0 / jnp.maximum(deg, 1.0)
    x = jnp.dot(agg * inv, wl_ref[...], preferred_element_type=jnp.float32)
    x = x + jnp.dot(emb_ref[...], wr_ref[...], preferred_element_type=jnp.float32)
    o_ref[...] = jnp.maximum(x + b_ref[...], 0.0)


_sage1_tc = pl.pallas_call(
    _sage1_body,
    grid=(_GRID,),
    in_specs=[
        pl.BlockSpec((NC, _BLK, D), lambda i: (0, i, 0)),
        pl.BlockSpec((NC, _BLK, D), lambda i: (0, i, 0)),
        pl.BlockSpec((_BLK, D), lambda i: (i, 0)),
        pl.BlockSpec((D, D), lambda i: (0, 0)),
        pl.BlockSpec((D, D), lambda i: (0, 0)),
        pl.BlockSpec((1, D), lambda i: (0, 0)),
    ],
    out_specs=pl.BlockSpec((_BLK, D), lambda i: (i, 0)),
    out_shape=jax.ShapeDtypeStruct((N_PAD, D), jnp.float32),
)


def _sage2_body(acc_ref, deg_ref, x1_ref, emb_ref, wl_ref, wr_ref, b_ref,
                o_ref):
    agg = acc_ref[0] + acc_ref[1]
    deg = deg_ref[0] + deg_ref[1]
    inv = 1.0 / jnp.maximum(deg, 1.0)
    x = jnp.dot(agg * inv, wl_ref[...], preferred_element_type=jnp.float32)
    x = x + jnp.dot(x1_ref[...], wr_ref[...],
                    preferred_element_type=jnp.float32)
    o_ref[...] = x + b_ref[...] + emb_ref[...]


_sage2_tc = pl.pallas_call(
    _sage2_body,
    grid=(_GRID,),
    in_specs=[
        pl.BlockSpec((NC, _BLK, D), lambda i: (0, i, 0)),
        pl.BlockSpec((NC, _BLK, D), lambda i: (0, i, 0)),
        pl.BlockSpec((_BLK, D), lambda i: (i, 0)),
        pl.BlockSpec((_BLK, D), lambda i: (i, 0)),
        pl.BlockSpec((D, D), lambda i: (0, 0)),
        pl.BlockSpec((D, D), lambda i: (0, 0)),
        pl.BlockSpec((1, D), lambda i: (0, 0)),
    ],
    out_specs=pl.BlockSpec((_BLK, D), lambda i: (i, 0)),
    out_shape=jax.ShapeDtypeStruct((N_PAD, D), jnp.float32),
)


_BB = 256
_BGRID = 1024 // _BB
_BN_SCALE = 1.0 / math.sqrt(1.0 + 1e-5)


def _head_body(s_ref, gamma_ref, beta_ref, f1w_ref, f1b_ref, f2w_ref,
               f2b_ref, o_ref):
    emd = jnp.concatenate([s_ref[0], s_ref[1]], axis=1)
    emd = (emd * _BN_SCALE) * gamma_ref[...] + beta_ref[...]
    h1 = jnp.dot(emd, f1w_ref[...], preferred_element_type=jnp.float32)
    h1 = jnp.maximum(h1 + f1b_ref[...], 0.0)
    h2 = jnp.dot(h1, f2w_ref[...], preferred_element_type=jnp.float32)
    o_ref[...] = h2 + f2b_ref[...]


_head_tc = pl.pallas_call(
    _head_body,
    grid=(_BGRID,),
    in_specs=[
        pl.BlockSpec((2, _BB, D), lambda i: (0, i, 0)),
        pl.BlockSpec((1, 2 * D), lambda i: (0, 0)),
        pl.BlockSpec((1, 2 * D), lambda i: (0, 0)),
        pl.BlockSpec((2 * D, 512), lambda i: (0, 0)),
        pl.BlockSpec((1, 512), lambda i: (0, 0)),
        pl.BlockSpec((512, D), lambda i: (0, 0)),
        pl.BlockSpec((1, D), lambda i: (0, 0)),
    ],
    out_specs=pl.BlockSpec((_BB, D), lambda i: (i, 0)),
    out_shape=jax.ShapeDtypeStruct((1024, D), jnp.float32),
)


def kernel(sentence, context, edge_index, emb, W1l, W1r, b1, W2l, W2r, b2,
           gamma, beta, fc1_w, fc1_b, fc2_w, fc2_b):
    edge_flat = edge_index.reshape(-1)
    zeros_buf = jnp.zeros((ROWS_PER_TILE, D), jnp.float32)
    ones_buf = jnp.ones((CH, D), jnp.float32)

    deg, acc1 = _deg_agg_kernel(edge_flat, emb, ones_buf, zeros_buf)
    deg = deg.reshape(NC, N_PAD, D)
    acc1 = acc1.reshape(NC, N_PAD, D)
    x1 = _sage1_tc(acc1, deg, emb, W1l, W1r, b1.reshape(1, D))

    acc2 = _agg_kernel(edge_flat, x1, zeros_buf).reshape(NC, N_PAD, D)
    x = _sage2_tc(acc2, deg, x1, emb, W2l, W2r, b2.reshape(1, D))

    idx_all = jnp.concatenate([sentence.reshape(-1),
                               context.reshape(-1)]).astype(jnp.int32)
    sums = _gsum_kernel(idx_all, x).reshape(2, 1024, D)

    fc2_pad = jnp.pad(fc2_w, ((0, 0), (0, D - 2)))
    fc2b_pad = jnp.pad(fc2_b, (0, D - 2)).reshape(1, D)
    out = _head_tc(sums, gamma.reshape(1, 2 * D), beta.reshape(1, 2 * D),
                   fc1_w, fc1_b.reshape(1, 512), fc2_pad, fc2b_pad)
    return out[:, :2]


# speedup vs baseline: 1.0017x; 1.0017x over previous
import functools
import math

import jax
import jax.numpy as jnp
from jax import lax
from jax.experimental import pallas as pl
from jax.experimental.pallas import tpu as pltpu
from jax.experimental.pallas import tpu_sc as plsc

N_NODES = 10000
N_PAD = 10240
N_EDGES = 320000
D = 128

NC = 2
NS = 16
NW = NC * NS
E_PER_W = N_EDGES // NW
ROWS_PER_TILE = N_PAD // NS

_mesh = plsc.VectorSubcoreMesh(core_axis_name="c", subcore_axis_name="s")


CH = 80
NCH = E_PER_W // CH
DEG_K = 25
N_STEADY = 120

_AGG_SCRATCH = [
    pltpu.VMEM_SHARED((N_PAD, D), jnp.float32),
    pltpu.VMEM((NCH, CH), jnp.int32),
    [pltpu.VMEM((CH,), jnp.int32) for _ in range(4)],
    [pltpu.VMEM((CH, D), jnp.float32) for _ in range(3)],
    [pltpu.SemaphoreType.DMA for _ in range(4)],
    [pltpu.SemaphoreType.DMA for _ in range(2)],
    [pltpu.SemaphoreType.DMA for _ in range(2)],
]


def _load_didx2(edge_hbm, didx2, base, sem):
    dbase = N_EDGES + base
    for lo in range(0, NCH, DEG_K):
        hi = min(lo + DEG_K, NCH)
        cps = [pltpu.async_copy(edge_hbm.at[pl.ds(dbase + r * CH, CH)],
                                didx2.at[r], sem)
               for r in range(lo, hi)]
        for cp in cps:
            cp.wait()


def _zero_acc(zeros_hbm, acc_sh, r0):
    pltpu.sync_copy(zeros_hbm, acc_sh.at[pl.ds(r0, ROWS_PER_TILE)])


def _agg_edge_loop(edge_hbm, x_hbm, acc_sh, didx2, sib, rows,
                   isem, gsem, ssem, base):
    def idxload(i, u4):
        pltpu.async_copy(edge_hbm.at[pl.ds(base + i * CH, CH)],
                         sib[u4], isem[u4])

    def idxwait(i, u4):
        pltpu.make_async_copy(edge_hbm.at[pl.ds(base + i * CH, CH)],
                              sib[u4], isem[u4]).wait()

    def gissue(u4, u3, u2):
        pltpu.async_copy(x_hbm.at[sib[u4]], rows[u3], gsem[u2])

    def gwait(u4, u3, u2):
        pltpu.make_async_copy(x_hbm.at[sib[u4]], rows[u3], gsem[u2]).wait()

    def sissue(i, u3, u2):
        pltpu.async_copy(rows[u3], acc_sh.at[didx2.at[i]], ssem[u2], add=True)

    def swait(i, u3, u2):
        pltpu.make_async_copy(rows[u3], acc_sh.at[didx2.at[i]],
                              ssem[u2]).wait()

    idxload(0, 0)
    idxload(1, 1)

    def body(t, carry):
        for u in range(12):
            j = t * 12 + u
            u4, u3, u2 = u % 4, u % 3, u % 2

            @pl.when(j >= 3)
            def _():
                swait(j - 3, u % 3, (u + 1) % 2)

            idxwait(j, u4)
            gissue(u4, u3, u2)

            @pl.when(j + 2 < NCH)
            def _():
                idxload(j + 2, (u + 2) % 4)

            @pl.when(j >= 1)
            def _():
                gwait((u + 3) % 4, (u + 2) % 3, (u + 1) % 2)
                sissue(j - 1, (u + 2) % 3, (u + 1) % 2)
        return carry

    lax.fori_loop(0, N_STEADY // 12, body, 0)

    for j in range(N_STEADY, NCH):
        u = j % 12
        u4, u3, u2 = u % 4, u % 3, u % 2
        swait(j - 3, u % 3, (u + 1) % 2)
        idxwait(j, u4)
        gissue(u4, u3, u2)
        if j + 2 < NCH:
            idxload(j + 2, (u + 2) % 4)
        gwait((u + 3) % 4, (u + 2) % 3, (u + 1) % 2)
        sissue(j - 1, (u + 2) % 3, (u + 1) % 2)
    uL = (NCH - 1) % 12
    gwait(uL % 4, uL % 3, uL % 2)
    sissue(NCH - 1, uL % 3, uL % 2)
    for j in range(NCH - 3, NCH):
        u = j % 12
        swait(j, u % 3, u % 2)


@functools.partial(
    pl.kernel,
    out_type=jax.ShapeDtypeStruct((NC * N_PAD, D), jnp.float32),
    mesh=_mesh,
    scratch_types=_AGG_SCRATCH,
)
def _agg_kernel(edge_hbm, x_hbm, zeros_hbm, acc_out,
                acc_sh, didx2, sib, rows, isem, gsem, ssem):
    c = lax.axis_index("c")
    s = lax.axis_index("s")
    wid = s * NC + c
    r0 = s * ROWS_PER_TILE
    base = wid * E_PER_W

    _zero_acc(zeros_hbm, acc_sh, r0)
    _load_didx2(edge_hbm, didx2, base, isem[3])
    plsc.subcore_barrier()
    _agg_edge_loop(edge_hbm, x_hbm, acc_sh, didx2, sib, rows,
                   isem, gsem, ssem, base)
    plsc.subcore_barrier()
    pltpu.sync_copy(acc_sh.at[pl.ds(r0, ROWS_PER_TILE)],
                    acc_out.at[pl.ds(c * N_PAD + r0, ROWS_PER_TILE)])


@functools.partial(
    pl.kernel,
    out_type=[jax.ShapeDtypeStruct((NC * N_PAD, D), jnp.float32),
              jax.ShapeDtypeStruct((NC * N_PAD, D), jnp.float32)],
    mesh=_mesh,
    scratch_types=_AGG_SCRATCH,
)
def _deg_agg_kernel(edge_hbm, x_hbm, ones_hbm, zeros_hbm,
                    deg_out, acc_out,
                    acc_sh, didx2, sib, rows, isem, gsem, ssem):
    c = lax.axis_index("c")
    s = lax.axis_index("s")
    wid = s * NC + c
    r0 = s * ROWS_PER_TILE
    base = wid * E_PER_W

    _zero_acc(zeros_hbm, acc_sh, r0)
    _load_didx2(edge_hbm, didx2, base, isem[3])
    pltpu.sync_copy(ones_hbm, rows[0])
    plsc.subcore_barrier()
    prev = None
    for bi, lo in enumerate(range(0, NCH, DEG_K)):
        hi = min(lo + DEG_K, NCH)
        cps = [pltpu.async_copy(rows[0], acc_sh.at[didx2.at[r]],
                                ssem[bi % 2], add=True)
               for r in range(lo, hi)]
        if prev is not None:
            for cp in prev:
                cp.wait()
        prev = cps
    for cp in prev:
        cp.wait()
    plsc.subcore_barrier()
    pltpu.sync_copy(acc_sh.at[pl.ds(r0, ROWS_PER_TILE)],
                    deg_out.at[pl.ds(c * N_PAD + r0, ROWS_PER_TILE)])
    plsc.subcore_barrier()

    _zero_acc(zeros_hbm, acc_sh, r0)
    plsc.subcore_barrier()
    _agg_edge_loop(edge_hbm, x_hbm, acc_sh, didx2, sib, rows,
                   isem, gsem, ssem, base)
    plsc.subcore_barrier()
    pltpu.sync_copy(acc_sh.at[pl.ds(r0, ROWS_PER_TILE)],
                    acc_out.at[pl.ds(c * N_PAD + r0, ROWS_PER_TILE)])


N_SUM_ROWS = 2048
L_CTX = 50
IDX_PER_TILE = N_SUM_ROWS * L_CTX // NW
GROUP = 4
N_PAIRS = 8


@functools.partial(
    pl.kernel,
    out_type=jax.ShapeDtypeStruct((N_SUM_ROWS, D), jnp.float32),
    mesh=_mesh,
    scratch_types=[
        pltpu.VMEM((IDX_PER_TILE,), jnp.int32),
        [pltpu.VMEM((GROUP * L_CTX, D), jnp.float32) for _ in range(2)],
        pltpu.VMEM((2 * GROUP, D), jnp.float32),
        [pltpu.SemaphoreType.DMA for _ in range(2)],
    ],
)
def _gsum_kernel(idx_hbm, x_hbm, out_hbm, idxv, rows, outv, sem):
    c = lax.axis_index("c")
    s = lax.axis_index("s")
    wid = s * NC + c
    pltpu.sync_copy(idx_hbm.at[pl.ds(wid * IDX_PER_TILE, IDX_PER_TILE)], idxv)

    def issue(h, b):
        ib = h * (GROUP * L_CTX)
        pltpu.async_copy(x_hbm.at[idxv.at[pl.ds(ib, 128)]],
                         rows[b].at[pl.ds(0, 128)], sem[b])
        pltpu.async_copy(x_hbm.at[idxv.at[pl.ds(ib + 128, 72)]],
                         rows[b].at[pl.ds(128, 72)], sem[b])

    def drain(h, b):
        ib = h * (GROUP * L_CTX)
        pltpu.make_async_copy(x_hbm.at[idxv.at[pl.ds(ib, 128)]],
                              rows[b].at[pl.ds(0, 128)], sem[b]).wait()
        pltpu.make_async_copy(x_hbm.at[idxv.at[pl.ds(ib + 128, 72)]],
                              rows[b].at[pl.ds(128, 72)], sem[b]).wait()

    def reduce(b, half):
        for o in range(GROUP):
            def jbody(j, accs):
                new = accs
                for u in range(5):
                    r = o * L_CTX + j * 5 + u
                    new = tuple(new[k] + rows[b][r, pl.ds(k * 16, 16)]
                                for k in range(8))
                return new
            init = tuple(jnp.zeros((16,), jnp.float32) for _ in range(8))
            accs = lax.fori_loop(0, 10, jbody, init)
            for k in range(8):
                outv[half * GROUP + o, pl.ds(k * 16, 16)] = accs[k]

    issue(0, 0)

    def body(p, carry):
        issue(2 * p + 1, 1)
        drain(2 * p, 0)
        reduce(0, 0)

        @pl.when(p < N_PAIRS - 1)
        def _():
            issue(2 * p + 2, 0)

        drain(2 * p + 1, 1)
        reduce(1, 1)
        pltpu.sync_copy(outv,
                        out_hbm.at[pl.ds(wid * 64 + p * (2 * GROUP),
                                         2 * GROUP)])
        return carry

    lax.fori_loop(0, N_PAIRS, body, 0)


_BLK = 1280
_GRID = N_PAD // _BLK


def _sage1_body(acc_ref, deg_ref, emb_ref, wl_ref, wr_ref, b_ref, o_ref):
    agg = acc_ref[0] + acc_ref[1]
    deg = deg_ref[0] + deg_ref[1]
    inv = 1.0 / jnp.maximum(deg, 1.0)
    x = jnp.dot(agg * inv, wl_ref[...], preferred_element_type=jnp.float32)
    x = x + jnp.dot(emb_ref[...], wr_ref[...], preferred_element_type=jnp.float32)
    o_ref[...] = jnp.maximum(x + b_ref[...], 0.0)


_sage1_tc = pl.pallas_call(
    _sage1_body,
    grid=(_GRID,),
    in_specs=[
        pl.BlockSpec((NC, _BLK, D), lambda i: (0, i, 0)),
        pl.BlockSpec((NC, _BLK, D), lambda i: (0, i, 0)),
        pl.BlockSpec((_BLK, D), lambda i: (i, 0)),
        pl.BlockSpec((D, D), lambda i: (0, 0)),
        pl.BlockSpec((D, D), lambda i: (0, 0)),
        pl.BlockSpec((1, D), lambda i: (0, 0)),
    ],
    out_specs=pl.BlockSpec((_BLK, D), lambda i: (i, 0)),
    out_shape=jax.ShapeDtypeStruct((N_PAD, D), jnp.float32),
)


def _sage2_body(acc_ref, deg_ref, x1_ref, emb_ref, wl_ref, wr_ref, b_ref,
                o_ref):
    agg = acc_ref[0] + acc_ref[1]
    deg = deg_ref[0] + deg_ref[1]
    inv = 1.0 / jnp.maximum(deg, 1.0)
    x = jnp.dot(agg * inv, wl_ref[...], preferred_element_type=jnp.float32)
    x = x + jnp.dot(x1_ref[...], wr_ref[...],
                    preferred_element_type=jnp.float32)
    o_ref[...] = x + b_ref[...] + emb_ref[...]


_sage2_tc = pl.pallas_call(
    _sage2_body,
    grid=(_GRID,),
    in_specs=[
        pl.BlockSpec((NC, _BLK, D), lambda i: (0, i, 0)),
        pl.BlockSpec((NC, _BLK, D), lambda i: (0, i, 0)),
        pl.BlockSpec((_BLK, D), lambda i: (i, 0)),
        pl.BlockSpec((_BLK, D), lambda i: (i, 0)),
        pl.BlockSpec((D, D), lambda i: (0, 0)),
        pl.BlockSpec((D, D), lambda i: (0, 0)),
        pl.BlockSpec((1, D), lambda i: (0, 0)),
    ],
    out_specs=pl.BlockSpec((_BLK, D), lambda i: (i, 0)),
    out_shape=jax.ShapeDtypeStruct((N_PAD, D), jnp.float32),
)


_BB = 256
_BGRID = 1024 // _BB
_BN_SCALE = 1.0 / math.sqrt(1.0 + 1e-5)


def _head_body(s_ref, gamma_ref, beta_ref, f1w_ref, f1b_ref, f2w_ref,
               f2b_ref, o_ref):
    emd = jnp.concatenate([s_ref[0], s_ref[1]], axis=1)
    emd = (emd * _BN_SCALE) * gamma_ref[...] + beta_ref[...]
    h1 = jnp.dot(emd, f1w_ref[...], preferred_element_type=jnp.float32)
    h1 = jnp.maximum(h1 + f1b_ref[...], 0.0)
    h2 = jnp.dot(h1, f2w_ref[...], preferred_element_type=jnp.float32)
    o_ref[...] = h2 + f2b_ref[...]


_head_tc = pl.pallas_call(
    _head_body,
    grid=(_BGRID,),
    in_specs=[
        pl.BlockSpec((2, _BB, D), lambda i: (0, i, 0)),
        pl.BlockSpec((1, 2 * D), lambda i: (0, 0)),
        pl.BlockSpec((1, 2 * D), lambda i: (0, 0)),
        pl.BlockSpec((2 * D, 512), lambda i: (0, 0)),
        pl.BlockSpec((1, 512), lambda i: (0, 0)),
        pl.BlockSpec((512, D), lambda i: (0, 0)),
        pl.BlockSpec((1, D), lambda i: (0, 0)),
    ],
    out_specs=pl.BlockSpec((_BB, D), lambda i: (i, 0)),
    out_shape=jax.ShapeDtypeStruct((1024, D), jnp.float32),
)


def kernel(sentence, context, edge_index, emb, W1l, W1r, b1, W2l, W2r, b2,
           gamma, beta, fc1_w, fc1_b, fc2_w, fc2_b):
    edge_flat = edge_index.reshape(-1)
    zeros_buf = jnp.zeros((ROWS_PER_TILE, D), jnp.float32)
    ones_buf = jnp.ones((CH, D), jnp.float32)

    deg, acc1 = _deg_agg_kernel(edge_flat, emb, ones_buf, zeros_buf)
    deg = deg.reshape(NC, N_PAD, D)
    acc1 = acc1.reshape(NC, N_PAD, D)
    x1 = _sage1_tc(acc1, deg, emb, W1l, W1r, b1.reshape(1, D))

    acc2 = _agg_kernel(edge_flat, x1, zeros_buf).reshape(NC, N_PAD, D)
    x = _sage2_tc(acc2, deg, x1, emb, W2l, W2r, b2.reshape(1, D))

    idx_all = jnp.concatenate([sentence.reshape(-1),
                               context.reshape(-1)]).astype(jnp.int32)
    sums = _gsum_kernel(idx_all, x).reshape(2, 1024, D)

    fc2_pad = jnp.pad(fc2_w, ((0, 0), (0, D - 2)))
    fc2b_pad = jnp.pad(fc2_b, (0, D - 2)).reshape(1, D)
    out = _head_tc(sums, gamma.reshape(1, 2 * D), beta.reshape(1, 2 * D),
                   fc1_w, fc1_b.reshape(1, 512), fc2_pad, fc2b_pad)
    return out[:, :2]
